# Initial kernel scaffold; baseline (speedup 1.0000x reference)
import functools

import jax
import jax.numpy as jnp
from jax import lax
from jax.experimental import pallas as pl
from jax.experimental.pallas import tpu as pltpu
from jax.experimental.pallas import tpu_sc as plsc

N_NODES = 10000
D = 128
N_EDGES = 320000

NC = 2
NS = 16
NW = NC * NS
EDGES_PER_TILE = N_EDGES // NW
CHUNK = 80
N_CHUNKS = EDGES_PER_TILE // CHUNK
ROWS_PER_TILE = N_NODES // NS

_mesh = plsc.VectorSubcoreMesh(
    core_axis_name="c", subcore_axis_name="s", num_cores=NC, num_subcores=NS
)


@functools.partial(
    pl.kernel,
    out_type=jax.ShapeDtypeStruct((NC, N_NODES), jnp.float32),
    mesh=_mesh,
    scratch_types=[
        pltpu.VMEM((CHUNK,), jnp.int32),
        pltpu.VMEM((CHUNK,), jnp.float32),
        pltpu.VMEM_SHARED((N_NODES,), jnp.float32),
        pltpu.SemaphoreType.DMA,
    ],
)
def _deg_kernel(dst_hbm, ones_hbm, zeros_hbm, out_hbm, idx_v, ones_v, hist_s, sem):
    c = lax.axis_index("c")
    s = lax.axis_index("s")
    wid = c * NS + s

    @pl.when(s < 10)
    def _():
        pltpu.sync_copy(zeros_hbm.at[pl.ds(s * 1000, 1000)],
                        hist_s.at[pl.ds(s * 1000, 1000)])

    pltpu.sync_copy(ones_hbm, ones_v)
    plsc.subcore_barrier()

    base = wid * EDGES_PER_TILE

    def body(j, carry):
        pltpu.sync_copy(dst_hbm.at[pl.ds(base + j * CHUNK, CHUNK)], idx_v)
        pltpu.sync_copy(ones_v, hist_s.at[idx_v], add=True)
        return carry

    lax.fori_loop(0, N_CHUNKS, body, 0)
    plsc.subcore_barrier()

    @pl.when(s < 10)
    def _():
        pltpu.sync_copy(hist_s.at[pl.ds(s * 1000, 1000)],
                        out_hbm.at[c, pl.ds(s * 1000, 1000)])


@functools.partial(
    pl.kernel,
    out_type=jax.ShapeDtypeStruct((NC, N_NODES, D), jnp.float32),
    mesh=_mesh,
    scratch_types=[
        pltpu.VMEM((CHUNK,), jnp.int32),
        pltpu.VMEM((CHUNK,), jnp.int32),
        pltpu.VMEM((CHUNK, D), jnp.float32),
        pltpu.VMEM_SHARED((N_NODES, D), jnp.float32),
        pltpu.SemaphoreType.DMA,
    ],
)
def _agg_kernel(y_hbm, src_hbm, dst_hbm, zeros_hbm, out_hbm,
                sidx_v, didx_v, rows_v, hacc_s, sem):
    c = lax.axis_index("c")
    s = lax.axis_index("s")
    wid = c * NS + s

    pltpu.sync_copy(zeros_hbm, hacc_s.at[pl.ds(s * ROWS_PER_TILE, ROWS_PER_TILE)])
    plsc.subcore_barrier()

    base = wid * EDGES_PER_TILE

    def body(j, carry):
        off = base + j * CHUNK
        pltpu.sync_copy(src_hbm.at[pl.ds(off, CHUNK)], sidx_v)
        pltpu.sync_copy(dst_hbm.at[pl.ds(off, CHUNK)], didx_v)
        pltpu.async_copy(y_hbm.at[sidx_v], rows_v, sem).wait()
        pltpu.sync_copy(rows_v, hacc_s.at[didx_v], add=True)
        return carry

    lax.fori_loop(0, N_CHUNKS, body, 0)
    plsc.subcore_barrier()

    pltpu.sync_copy(hacc_s.at[pl.ds(s * ROWS_PER_TILE, ROWS_PER_TILE)],
                    out_hbm.at[c, pl.ds(s * ROWS_PER_TILE, ROWS_PER_TILE)])


def _mm_body(x_ref, w_ref, degt_ref, y_ref, dinv_ref):
    xw = jnp.dot(x_ref[...], w_ref[...], preferred_element_type=jnp.float32)
    deg = degt_ref[:, 0:1] + degt_ref[:, 1:2] + 1.0
    dinv = lax.rsqrt(deg)
    dinv_ref[...] = dinv
    y_ref[...] = xw * dinv


_mm = pl.pallas_call(
    _mm_body,
    out_shape=[
        jax.ShapeDtypeStruct((N_NODES, D), jnp.float32),
        jax.ShapeDtypeStruct((N_NODES, 1), jnp.float32),
    ],
)


def _fin_body(h0_ref, h1_ref, y_ref, dinv_ref, x_ref, b_ref, g_ref, be_ref, o_ref):
    h = (h0_ref[...] + h1_ref[...] + y_ref[...]) * dinv_ref[...] + b_ref[...]
    h = jnp.maximum(h, 0.0) + x_ref[...]
    m = jnp.mean(h, axis=0, keepdims=True)
    d = h - m
    v = jnp.mean(d * d, axis=0, keepdims=True)
    o_ref[...] = d * lax.rsqrt(v + 1e-5) * g_ref[...] + be_ref[...]


_fin = pl.pallas_call(
    _fin_body,
    out_shape=jax.ShapeDtypeStruct((N_NODES, D), jnp.float32),
)


def kernel(x, edge_index, W, b, gamma, beta):
    ei = edge_index.astype(jnp.int32)
    src = ei[0]
    dst = ei[1]
    ones_c = jnp.ones((CHUNK,), jnp.float32)
    zeros_n = jnp.zeros((N_NODES,), jnp.float32)
    deg_part = _deg_kernel(dst, ones_c, zeros_n)
    degt = deg_part.T
    y, dinv = _mm(x, W, degt)
    zeros_rows = jnp.zeros((ROWS_PER_TILE, D), jnp.float32)
    h_part = _agg_kernel(y, src, dst, zeros_rows)
    return _fin(h_part[0], h_part[1], y, dinv, x,
                b.reshape(1, D), gamma.reshape(1, D), beta.reshape(1, D))


# speedup vs baseline: 15.6054x; 15.6054x over previous
import functools

import jax
import jax.numpy as jnp
from jax import lax
from jax.experimental import pallas as pl
from jax.experimental.pallas import tpu as pltpu
from jax.experimental.pallas import tpu_sc as plsc

N_NODES = 10000
D = 128
N_EDGES = 320000

NC = 2
NS = 16
NW = NC * NS
EDGES_PER_TILE = N_EDGES // NW
CHUNK = 80
N_CHUNKS = EDGES_PER_TILE // CHUNK
N_PAD = 10240
ROWS_PER_TILE = N_PAD // NS

_mesh = plsc.VectorSubcoreMesh(
    core_axis_name="c", subcore_axis_name="s", num_cores=NC, num_subcores=NS
)


@functools.partial(
    pl.kernel,
    out_type=jax.ShapeDtypeStruct((NC * N_NODES,), jnp.float32),
    mesh=_mesh,
    scratch_types=[
        pltpu.VMEM((CHUNK,), jnp.int32),
        pltpu.VMEM((CHUNK,), jnp.float32),
        pltpu.VMEM((1000,), jnp.float32),
        pltpu.VMEM_SHARED((N_NODES,), jnp.float32),
        pltpu.SemaphoreType.DMA,
    ],
)
def _deg_kernel(dst_hbm, ones_hbm, zeros_hbm, out_hbm, idx_v, ones_v, stage_v,
                hist_s, sem):
    c = lax.axis_index("c")
    s = lax.axis_index("s")
    wid = c * NS + s

    @pl.when(s < 10)
    def _():
        pltpu.sync_copy(zeros_hbm, stage_v)
        pltpu.sync_copy(stage_v, hist_s.at[pl.ds(s * 1000, 1000)])

    pltpu.sync_copy(ones_hbm, ones_v)
    plsc.subcore_barrier()

    base = wid * EDGES_PER_TILE

    def body(j, carry):
        pltpu.sync_copy(dst_hbm.at[pl.ds(base + j * CHUNK, CHUNK)], idx_v)
        pltpu.sync_copy(ones_v, hist_s.at[idx_v], add=True)
        return carry

    lax.fori_loop(0, N_CHUNKS, body, 0)
    plsc.subcore_barrier()

    @pl.when(s < 10)
    def _():
        pltpu.sync_copy(hist_s.at[pl.ds(s * 1000, 1000)], stage_v)
        pltpu.sync_copy(stage_v, out_hbm.at[pl.ds(c * N_NODES + s * 1000, 1000)])


@functools.partial(
    pl.kernel,
    out_type=jax.ShapeDtypeStruct((NC, N_PAD, D), jnp.float32),
    mesh=_mesh,
    scratch_types=[
        pltpu.VMEM((CHUNK,), jnp.int32),
        pltpu.VMEM((CHUNK,), jnp.int32),
        pltpu.VMEM((CHUNK, D), jnp.float32),
        pltpu.VMEM_SHARED((N_PAD, D), jnp.float32),
        pltpu.SemaphoreType.DMA,
    ],
)
def _agg_kernel(y_hbm, src_hbm, dst_hbm, zeros_hbm, out_hbm,
                sidx_v, didx_v, rows_v, hacc_s, sem):
    c = lax.axis_index("c")
    s = lax.axis_index("s")
    wid = c * NS + s
    row0 = s * ROWS_PER_TILE
    n_sub = ROWS_PER_TILE // CHUNK

    pltpu.sync_copy(zeros_hbm, rows_v)

    def zbody(i, carry):
        pltpu.sync_copy(rows_v, hacc_s.at[pl.ds(row0 + i * CHUNK, CHUNK)])
        return carry

    lax.fori_loop(0, n_sub, zbody, 0)
    plsc.subcore_barrier()

    base = wid * EDGES_PER_TILE

    def body(j, carry):
        off = base + j * CHUNK
        pltpu.sync_copy(src_hbm.at[pl.ds(off, CHUNK)], sidx_v)
        pltpu.sync_copy(dst_hbm.at[pl.ds(off, CHUNK)], didx_v)
        pltpu.async_copy(y_hbm.at[sidx_v], rows_v, sem).wait()
        pltpu.sync_copy(rows_v, hacc_s.at[didx_v], add=True)
        return carry

    lax.fori_loop(0, N_CHUNKS, body, 0)
    plsc.subcore_barrier()

    def obody(i, carry):
        r = row0 + i * CHUNK
        pltpu.sync_copy(hacc_s.at[pl.ds(r, CHUNK)], rows_v)
        pltpu.sync_copy(rows_v, out_hbm.at[c, pl.ds(r, CHUNK)])
        return carry

    lax.fori_loop(0, n_sub, obody, 0)


def _mm_body(x_ref, w_ref, degt_ref, y_ref, dinv_ref):
    xw = jnp.dot(x_ref[...], w_ref[...], preferred_element_type=jnp.float32)
    deg = degt_ref[:, 0:1] + degt_ref[:, 1:2] + 1.0
    dinv = lax.rsqrt(deg)
    dinv_ref[...] = dinv
    y_ref[...] = xw * dinv


_mm = pl.pallas_call(
    _mm_body,
    out_shape=[
        jax.ShapeDtypeStruct((N_NODES, D), jnp.float32),
        jax.ShapeDtypeStruct((N_NODES, 1), jnp.float32),
    ],
)


def _fin_body(h0_ref, h1_ref, y_ref, dinv_ref, x_ref, b_ref, g_ref, be_ref, o_ref):
    h = (h0_ref[...] + h1_ref[...] + y_ref[...]) * dinv_ref[...] + b_ref[...]
    h = jnp.maximum(h, 0.0) + x_ref[...]
    m = jnp.mean(h, axis=0, keepdims=True)
    d = h - m
    v = jnp.mean(d * d, axis=0, keepdims=True)
    o_ref[...] = d * lax.rsqrt(v + 1e-5) * g_ref[...] + be_ref[...]


_fin = pl.pallas_call(
    _fin_body,
    out_shape=jax.ShapeDtypeStruct((N_NODES, D), jnp.float32),
)


def kernel(x, edge_index, W, b, gamma, beta):
    ei = edge_index.astype(jnp.int32)
    src = ei[0]
    dst = ei[1]
    ones_c = jnp.ones((CHUNK,), jnp.float32)
    zeros_n = jnp.zeros((1000,), jnp.float32)
    deg_part = _deg_kernel(dst, ones_c, zeros_n)
    degt = deg_part.reshape(NC, N_NODES).T
    y, dinv = _mm(x, W, degt)
    zeros_rows = jnp.zeros((CHUNK, D), jnp.float32)
    h_part = _agg_kernel(y, src, dst, zeros_rows)
    return _fin(h_part[0, :N_NODES], h_part[1, :N_NODES], y, dinv, x,
                b.reshape(1, D), gamma.reshape(1, D), beta.reshape(1, D))


# speedup vs baseline: 29.5635x; 1.8944x over previous
import functools

import jax
import jax.numpy as jnp
from jax import lax
from jax.experimental import pallas as pl
from jax.experimental.pallas import tpu as pltpu
from jax.experimental.pallas import tpu_sc as plsc

N_NODES = 10000
D = 128
N_EDGES = 320000

NC = 2
NS = 16
NW = NC * NS
EDGES_PER_TILE = N_EDGES // NW
CHUNK = 80
N_CHUNKS = EDGES_PER_TILE // CHUNK
N_PAIRS = (N_CHUNKS - 1) // 2
N_PAD = 10240
ROWS_PER_TILE = N_PAD // NS
OUT_SUB = ROWS_PER_TILE // CHUNK

_mesh = plsc.VectorSubcoreMesh(
    core_axis_name="c", subcore_axis_name="s", num_cores=NC, num_subcores=NS
)


@functools.partial(
    pl.kernel,
    out_type=jax.ShapeDtypeStruct((NC * N_NODES,), jnp.float32),
    mesh=_mesh,
    scratch_types=[
        pltpu.VMEM((N_CHUNKS, CHUNK), jnp.int32),
        pltpu.VMEM((CHUNK,), jnp.float32),
        pltpu.VMEM((1000,), jnp.float32),
        pltpu.VMEM_SHARED((N_NODES,), jnp.float32),
        pltpu.SemaphoreType.DMA,
        pltpu.SemaphoreType.DMA,
    ],
)
def _deg_kernel(dst_hbm, ones_hbm, zeros_hbm, out_hbm, didx_m, ones_v, stage_v,
                hist_s, sem0, sem1):
    c = lax.axis_index("c")
    s = lax.axis_index("s")
    wid = c * NS + s

    @pl.when(s < 10)
    def _():
        pltpu.sync_copy(zeros_hbm, stage_v)
        pltpu.sync_copy(stage_v, hist_s.at[pl.ds(s * 1000, 1000)])

    pltpu.sync_copy(ones_hbm, ones_v)
    pltpu.sync_copy(dst_hbm.at[wid], didx_m)
    plsc.subcore_barrier()

    def scat_start(j, sem):
        pltpu.async_copy(ones_v, hist_s.at[didx_m.at[j]], sem, add=True)

    def scat_wait(sem):
        pltpu.make_async_copy(ones_v, hist_s.at[didx_m.at[0]], sem).wait()

    scat_start(0, sem0)
    scat_start(1, sem1)

    def pair(jj, carry):
        j = 2 * jj
        scat_wait(sem0)

        @pl.when(j + 2 < N_CHUNKS)
        def _():
            scat_start(j + 2, sem0)

        scat_wait(sem1)

        @pl.when(j + 3 < N_CHUNKS)
        def _():
            scat_start(j + 3, sem1)

        return carry

    lax.fori_loop(0, N_PAIRS, pair, 0)
    scat_wait(sem0)
    plsc.subcore_barrier()

    @pl.when(s < 10)
    def _():
        pltpu.sync_copy(hist_s.at[pl.ds(s * 1000, 1000)], stage_v)
        pltpu.sync_copy(stage_v, out_hbm.at[pl.ds(c * N_NODES + s * 1000, 1000)])


@functools.partial(
    pl.kernel,
    out_type=jax.ShapeDtypeStruct((NC, N_PAD, D), jnp.float32),
    mesh=_mesh,
    scratch_types=[
        pltpu.VMEM((EDGES_PER_TILE,), jnp.int32),
        pltpu.VMEM((N_CHUNKS, CHUNK), jnp.int32),
        pltpu.VMEM((CHUNK, D), jnp.float32),
        pltpu.VMEM((CHUNK, D), jnp.float32),
        pltpu.VMEM_SHARED((N_PAD, D), jnp.float32),
        pltpu.SemaphoreType.DMA,
        pltpu.SemaphoreType.DMA,
        pltpu.SemaphoreType.DMA,
        pltpu.SemaphoreType.DMA,
    ],
)
def _agg_kernel(y_hbm, src_hbm, dst_hbm, zeros_hbm, out_hbm,
                sidx_v, didx_m, rows0, rows1, hacc_s,
                gsem0, gsem1, ssem0, ssem1):
    c = lax.axis_index("c")
    s = lax.axis_index("s")
    wid = c * NS + s
    row0 = s * ROWS_PER_TILE

    pltpu.sync_copy(zeros_hbm, rows0)

    def zbody(i, carry):
        pltpu.sync_copy(rows0, hacc_s.at[pl.ds(row0 + i * CHUNK, CHUNK)])
        return carry

    lax.fori_loop(0, OUT_SUB, zbody, 0)

    pltpu.sync_copy(src_hbm.at[wid], sidx_v)
    pltpu.sync_copy(dst_hbm.at[wid], didx_m)

    def g_start(j, buf, sem):
        pltpu.async_copy(y_hbm.at[sidx_v.at[pl.ds(j * CHUNK, CHUNK)]], buf, sem)

    def g_wait(buf, sem):
        pltpu.make_async_copy(y_hbm.at[sidx_v.at[pl.ds(0, CHUNK)]], buf, sem).wait()

    def s_start(j, buf, sem):
        pltpu.async_copy(buf, hacc_s.at[didx_m.at[j]], sem, add=True)

    def s_wait(buf, sem):
        pltpu.make_async_copy(buf, hacc_s.at[didx_m.at[0]], sem).wait()

    g_start(0, rows0, gsem0)
    g_start(1, rows1, gsem1)
    plsc.subcore_barrier()

    def pair(jj, carry):
        j = 2 * jj
        g_wait(rows0, gsem0)
        s_start(j, rows0, ssem0)
        g_wait(rows1, gsem1)
        s_start(j + 1, rows1, ssem1)
        s_wait(rows0, ssem0)

        @pl.when(j + 2 < N_CHUNKS)
        def _():
            g_start(j + 2, rows0, gsem0)

        s_wait(rows1, ssem1)

        @pl.when(j + 3 < N_CHUNKS)
        def _():
            g_start(j + 3, rows1, gsem1)

        return carry

    lax.fori_loop(0, N_PAIRS, pair, 0)
    g_wait(rows0, gsem0)
    s_start(N_CHUNKS - 1, rows0, ssem0)
    s_wait(rows0, ssem0)
    plsc.subcore_barrier()

    def obody(i, carry):
        r = row0 + i * CHUNK
        pltpu.sync_copy(hacc_s.at[pl.ds(r, CHUNK)], rows0)
        pltpu.sync_copy(rows0, out_hbm.at[c, pl.ds(r, CHUNK)])
        return carry

    lax.fori_loop(0, OUT_SUB, obody, 0)


def _mm_body(x_ref, w_ref, degt_ref, y_ref, dinv_ref):
    xw = jnp.dot(x_ref[...], w_ref[...], preferred_element_type=jnp.float32)
    deg = degt_ref[:, 0:1] + degt_ref[:, 1:2] + 1.0
    dinv = lax.rsqrt(deg)
    dinv_ref[...] = dinv
    y_ref[...] = xw * dinv


_mm = pl.pallas_call(
    _mm_body,
    out_shape=[
        jax.ShapeDtypeStruct((N_NODES, D), jnp.float32),
        jax.ShapeDtypeStruct((N_NODES, 1), jnp.float32),
    ],
)


def _fin_body(h0_ref, h1_ref, y_ref, dinv_ref, x_ref, b_ref, g_ref, be_ref, o_ref):
    h = (h0_ref[...] + h1_ref[...] + y_ref[...]) * dinv_ref[...] + b_ref[...]
    h = jnp.maximum(h, 0.0) + x_ref[...]
    m = jnp.mean(h, axis=0, keepdims=True)
    d = h - m
    v = jnp.mean(d * d, axis=0, keepdims=True)
    o_ref[...] = d * lax.rsqrt(v + 1e-5) * g_ref[...] + be_ref[...]


_fin = pl.pallas_call(
    _fin_body,
    out_shape=jax.ShapeDtypeStruct((N_NODES, D), jnp.float32),
)


def kernel(x, edge_index, W, b, gamma, beta):
    ei = edge_index.astype(jnp.int32)
    src2 = ei[0].reshape(NW, EDGES_PER_TILE)
    dst3 = ei[1].reshape(NW, N_CHUNKS, CHUNK)
    ones_c = jnp.ones((CHUNK,), jnp.float32)
    zeros_n = jnp.zeros((1000,), jnp.float32)
    deg_part = _deg_kernel(dst3, ones_c, zeros_n)
    degt = deg_part.reshape(NC, N_NODES).T
    y, dinv = _mm(x, W, degt)
    zeros_rows = jnp.zeros((CHUNK, D), jnp.float32)
    h_part = _agg_kernel(y, src2, dst3, zeros_rows)
    return _fin(h_part[0, :N_NODES], h_part[1, :N_NODES], y, dinv, x,
                b.reshape(1, D), gamma.reshape(1, D), beta.reshape(1, D))
